# baseline (speedup 1.0000x reference)
import functools

import jax
import jax.numpy as jnp
from jax import lax
from jax.experimental import pallas as pl
from jax.experimental.pallas import tpu as pltpu
from jax.experimental.pallas import tpu_sc as plsc

N = 10000
E = 320000
T = 8
D = 128
GRU_H = 256

NC = 2
NS = 16
NW = NC * NS
EPW = E // NW
K = 128
SEG = 3
CPS = 28
NCH = SEG * CPS
EPW_PAD = NCH * K
RPT = N // NS
SINK = N

_mesh = plsc.VectorSubcoreMesh(core_axis_name="c", subcore_axis_name="s")
_sc_params = pltpu.CompilerParams(use_tc_tiling_on_sc=False,
                                  needs_layout_passes=False)


def _deg_body(dst_hbm, out_hbm, idx_v, hist_v, sem):
    c = lax.axis_index("c")
    s = lax.axis_index("s")
    w = c * NS + s

    @pl.loop(0, T)
    def _t(t):
        @pl.loop(0, RPT)
        def _z(i):
            hist_v[i, :] = jnp.zeros((16,), jnp.float32)

        pltpu.async_copy(dst_hbm.at[t, pl.ds(w * EPW, EPW)], idx_v, sem).wait()

        ones = jnp.ones((16,), jnp.float32)

        @pl.loop(0, EPW // 16)
        def _e(i):
            idx = idx_v[pl.ds(i * 16, 16)]
            hi = jax.lax.shift_right_logical(idx, 4)
            lo = jnp.bitwise_and(idx, jnp.full((16,), 15, jnp.int32))
            plsc.addupdate_scatter(hist_v, [hi, lo], ones)

        pltpu.async_copy(hist_v, out_hbm.at[w, t], sem).wait()


@jax.jit
def _sc_deg(dst):
    kern = pl.kernel(
        _deg_body,
        out_type=jax.ShapeDtypeStruct((NW, T, RPT, 16), jnp.float32),
        mesh=_mesh,
        compiler_params=_sc_params,
        scratch_types=[
            pltpu.VMEM((EPW,), jnp.int32),
            pltpu.VMEM((RPT, 16), jnp.float32),
            pltpu.SemaphoreType.DMA,
        ],
    )
    return kern(dst)


def _agg_body(y_hbm, sd_hbm, zeros_hbm, out_hbm, accum,
              idxb, rows, sem_i, sem_g, sem_s, sem_z):
    c = lax.axis_index("c")
    s = lax.axis_index("s")
    w = c * NS + s

    def gath_d(ib, gg, b):
        return pltpu.make_async_copy(y_hbm.at[idxb[ib].at[gg, 0]], rows[b],
                                     sem_g[b])

    def scat_d(ib, gg, b):
        return pltpu.make_async_copy(rows[b], accum.at[idxb[ib].at[gg, 1]],
                                     sem_s[b])

    def _segment(ib):
        gath_d(ib, 0, 0).start()
        @pl.loop(0, (CPS - 2) // 2)
        def _i(i):
            for b, off in ((1, 1), (0, 2)):
                gg = 2 * i + off

                @pl.when(gg >= 2)
                def _w0():
                    scat_d(ib, gg - 2, b).wait()

                gath_d(ib, gg, b).start()
                gath_d(ib, gg - 1, 1 - b).wait()
                pltpu.async_copy(rows[1 - b], accum.at[idxb[ib].at[gg - 1, 1]],
                                 sem_s[1 - b], add=True)

        gg = CPS - 1
        scat_d(ib, gg - 2, 1).wait()
        gath_d(ib, gg, 1).start()
        gath_d(ib, gg - 1, 0).wait()
        pltpu.async_copy(rows[0], accum.at[idxb[ib].at[gg - 1, 1]],
                         sem_s[0], add=True)
        gath_d(ib, gg, 1).wait()
        pltpu.async_copy(rows[1], accum.at[idxb[ib].at[gg, 1]],
                         sem_s[1], add=True)
        scat_d(ib, gg - 1, 0).wait()
        scat_d(ib, gg, 1).wait()

    @pl.loop(0, T)
    def _t(t):
        pltpu.async_copy(zeros_hbm, accum.at[pl.ds(s * RPT, RPT)], sem_z)
        pltpu.async_copy(sd_hbm.at[t, w, 0], idxb[0], sem_i[0])
        pltpu.make_async_copy(zeros_hbm, accum.at[pl.ds(s * RPT, RPT)],
                              sem_z).wait()
        pltpu.make_async_copy(sd_hbm.at[t, w, 0], idxb[0], sem_i[0]).wait()
        plsc.subcore_barrier()

        for sg in range(SEG):
            ib = sg % 2
            if sg + 1 < SEG:
                pltpu.async_copy(sd_hbm.at[t, w, sg + 1], idxb[1 - ib],
                                 sem_i[1 - ib])
            _segment(ib)
            if sg + 1 < SEG:
                pltpu.make_async_copy(sd_hbm.at[t, w, sg + 1], idxb[1 - ib],
                                      sem_i[1 - ib]).wait()

        plsc.subcore_barrier()
        pltpu.sync_copy(accum.at[pl.ds(s * RPT, RPT)],
                        out_hbm.at[c, t, pl.ds(s * RPT, RPT)])


def _agg_call(y, sd, zrows):
    kern = pl.kernel(
        _agg_body,
        out_type=jax.ShapeDtypeStruct((NC, T, N, D), jnp.float32),
        mesh=_mesh,
        compiler_params=_sc_params,
        scratch_types=[
            pltpu.VMEM_SHARED((N + 16, D), jnp.float32),
            [pltpu.VMEM((CPS, 2, K), jnp.int32) for _ in range(2)],
            [pltpu.VMEM((K, D), jnp.float32) for _ in range(2)],
            [pltpu.SemaphoreType.DMA for _ in range(2)],
            [pltpu.SemaphoreType.DMA for _ in range(2)],
            [pltpu.SemaphoreType.DMA for _ in range(2)],
            pltpu.SemaphoreType.DMA,
        ],
    )
    return kern(y, sd, zrows)


_RB = 1000
_NRB = T * N // _RB


def _tca_body(x_ref, w_ref, degp_ref, y_ref):
    deg = jnp.sum(degp_ref[...], axis=1) + 1.0
    dinv = jax.lax.rsqrt(deg)
    xw = jnp.dot(x_ref[...], w_ref[...], preferred_element_type=jnp.float32)
    y_ref[...] = xw * dinv[:, None]


@jax.jit
def _tc_a(x2d, W1, degpt):
    return pl.pallas_call(
        _tca_body,
        grid=(_NRB,),
        in_specs=[
            pl.BlockSpec((_RB, D), lambda i: (i, 0)),
            pl.BlockSpec((D, D), lambda i: (0, 0)),
            pl.BlockSpec((_RB, NW), lambda i: (i, 0)),
        ],
        out_specs=pl.BlockSpec((_RB, D), lambda i: (i, 0)),
        out_shape=jax.ShapeDtypeStruct((T * N, D), jnp.float32),
    )(x2d, W1, degpt)


def _tcb_body(aggp_ref, y1_ref, degp_ref, w_ref, b_ref, y2_ref):
    deg = jnp.sum(degp_ref[...], axis=1) + 1.0
    dinv = jax.lax.rsqrt(deg)[:, None]
    a = aggp_ref[0] + aggp_ref[1] + y1_ref[...]
    h1 = jnp.maximum(a * dinv + b_ref[...], 0.0)
    y2_ref[...] = jnp.dot(h1, w_ref[...],
                          preferred_element_type=jnp.float32) * dinv


@jax.jit
def _tc_b(aggp, y1, degpt, W2, b1):
    return pl.pallas_call(
        _tcb_body,
        grid=(_NRB,),
        in_specs=[
            pl.BlockSpec((NC, _RB, D), lambda i: (0, i, 0)),
            pl.BlockSpec((_RB, D), lambda i: (i, 0)),
            pl.BlockSpec((_RB, NW), lambda i: (i, 0)),
            pl.BlockSpec((D, D), lambda i: (0, 0)),
            pl.BlockSpec((1, D), lambda i: (0, 0)),
        ],
        out_specs=pl.BlockSpec((_RB, D), lambda i: (i, 0)),
        out_shape=jax.ShapeDtypeStruct((T * N, D), jnp.float32),
    )(aggp, y1, degpt, W2, b1)


def _tcc_body(aggp_ref, y2_ref, degp_ref, b_ref, emb_ref):
    j = pl.program_id(1)

    @pl.when(j == 0)
    def _():
        emb_ref[...] = jnp.zeros_like(emb_ref)

    deg = jnp.sum(degp_ref[...], axis=1) + 1.0
    dinv = jax.lax.rsqrt(deg)[:, None]
    a = aggp_ref[0] + aggp_ref[1] + y2_ref[...]
    h2 = jnp.maximum(a * dinv + b_ref[...], 0.0)
    emb_ref[0, 0:1, :] += jnp.sum(h2, axis=0, keepdims=True) * (1.0 / N)


@jax.jit
def _tc_c(aggp, y2, degpt, b2):
    npt = N // _RB

    return pl.pallas_call(
        _tcc_body,
        grid=(T, npt),
        in_specs=[
            pl.BlockSpec((NC, _RB, D), lambda t, j: (0, t * npt + j, 0)),
            pl.BlockSpec((_RB, D), lambda t, j: (t * npt + j, 0)),
            pl.BlockSpec((_RB, NW), lambda t, j: (t * npt + j, 0)),
            pl.BlockSpec((1, D), lambda t, j: (0, 0)),
        ],
        out_specs=pl.BlockSpec((1, 8, D), lambda t, j: (t, 0, 0)),
        out_shape=jax.ShapeDtypeStruct((T, 8, D), jnp.float32),
    )(aggp, y2, degpt, b2)


def _gru_body(emb_ref, wih_ref, whh_ref, bih_ref, bhh_ref, wc_ref, bc_ref,
              out_ref):
    h = jnp.zeros((1, GRU_H), jnp.float32)
    for t in range(T):
        xt = emb_ref[pl.ds(t, 1), :]
        gi = jnp.dot(xt, wih_ref[...],
                     preferred_element_type=jnp.float32) + bih_ref[...]
        gh = jnp.dot(h, whh_ref[...],
                     preferred_element_type=jnp.float32) + bhh_ref[...]
        r = jax.nn.sigmoid(gi[:, :GRU_H] + gh[:, :GRU_H])
        z = jax.nn.sigmoid(gi[:, GRU_H:2 * GRU_H] + gh[:, GRU_H:2 * GRU_H])
        n = jnp.tanh(gi[:, 2 * GRU_H:] + r * gh[:, 2 * GRU_H:])
        h = (1.0 - z) * n + z * h
    out_ref[...] = jnp.dot(h, wc_ref[...],
                           preferred_element_type=jnp.float32) + bc_ref[...]


@jax.jit
def _tc_gru(emb, wihT, whhT, bih, bhh, wcT, bc):
    return pl.pallas_call(
        _gru_body,
        out_shape=jax.ShapeDtypeStruct((1, D), jnp.float32),
    )(emb, wihT, whhT, bih, bhh, wcT, bc)


def kernel(x, edge_index, W1, b1, W2, b2, W_ih, W_hh, b_ih, b_hh, Wc, bc):
    ei = edge_index.astype(jnp.int32)
    src = ei[:, 0, :]
    dst = ei[:, 1, :]

    srcg = src + (jnp.arange(T, dtype=jnp.int32) * N)[:, None]
    pad = ((0, 0), (0, 0), (0, EPW_PAD - EPW))
    srcp = jnp.pad(srcg.reshape(T, NW, EPW), pad,
                   constant_values=0).reshape(T, NW, NCH, K)
    dstp = jnp.pad(dst.reshape(T, NW, EPW), pad,
                   constant_values=SINK).reshape(T, NW, NCH, K)
    sd = jnp.stack([srcp, dstp], axis=3).reshape(T, NW, SEG, CPS, 2, K)
    zrows = jnp.zeros((RPT, D), jnp.float32)

    degp = _sc_deg(dst)
    degpt = degp.reshape(NW, T * N).T

    x2d = x.reshape(T * N, D)
    y1 = _tc_a(x2d, W1, degpt)
    aggp1 = _agg_call(y1, sd, zrows)
    y2 = _tc_b(aggp1.reshape(NC, T * N, D), y1, degpt, W2, b1[None, :])
    aggp2 = _agg_call(y2, sd, zrows)
    emb = _tc_c(aggp2.reshape(NC, T * N, D), y2, degpt, b2[None, :])[:, 0, :]

    wcT = jnp.zeros((GRU_H, D), jnp.float32).at[:, :10].set(Wc.T)
    bcp = jnp.zeros((1, D), jnp.float32).at[0, :10].set(bc)
    logits = _tc_gru(emb, W_ih.T, W_hh.T, b_ih[None, :], b_hh[None, :],
                     wcT, bcp)
    return logits[:, :10]


# speedup vs baseline: 3.9612x; 3.9612x over previous
import functools

import jax
import jax.numpy as jnp
from jax import lax
from jax.experimental import pallas as pl
from jax.experimental.pallas import tpu as pltpu
from jax.experimental.pallas import tpu_sc as plsc

N = 10000
E = 320000
T = 8
D = 128
GRU_H = 256

NC = 2
NS = 16
NW = NC * NS
EPW = E // NW
K = 128
NCH = 80
EPW_PAD = NCH * K
RPT = N // NS
SINK = N
NBUF = 4
ISLOT = 8

_mesh = plsc.VectorSubcoreMesh(core_axis_name="c", subcore_axis_name="s")
_sc_params = pltpu.CompilerParams(use_tc_tiling_on_sc=False,
                                  needs_layout_passes=False)


def _deg_body(dst_hbm, out_hbm, idx_v, hist_v, sem):
    c = lax.axis_index("c")
    s = lax.axis_index("s")
    w = c * NS + s

    @pl.loop(0, T)
    def _t(t):
        @pl.loop(0, RPT)
        def _z(i):
            hist_v[i, :] = jnp.zeros((16,), jnp.float32)

        pltpu.async_copy(dst_hbm.at[t, pl.ds(w * EPW, EPW)], idx_v, sem).wait()

        ones = jnp.ones((16,), jnp.float32)

        @pl.loop(0, EPW // 16)
        def _e(i):
            idx = idx_v[pl.ds(i * 16, 16)]
            hi = jax.lax.shift_right_logical(idx, 4)
            lo = jnp.bitwise_and(idx, jnp.full((16,), 15, jnp.int32))
            plsc.addupdate_scatter(hist_v, [hi, lo], ones)

        pltpu.async_copy(hist_v, out_hbm.at[w, t], sem).wait()


@jax.jit
def _sc_deg(dst):
    kern = pl.kernel(
        _deg_body,
        out_type=jax.ShapeDtypeStruct((NW, T, RPT, 16), jnp.float32),
        mesh=_mesh,
        compiler_params=_sc_params,
        scratch_types=[
            pltpu.VMEM((EPW,), jnp.int32),
            pltpu.VMEM((RPT, 16), jnp.float32),
            pltpu.SemaphoreType.DMA,
        ],
    )
    return kern(dst)


def _agg_body(y_hbm, srcp_hbm, dstp_hbm, zeros_hbm, out_hbm, accum,
              isrc, idst, rows, sem_i, sem_g, sem_s, sem_z):
    c = lax.axis_index("c")
    s = lax.axis_index("s")
    w = c * NS + s

    def idx_pair(t, g, slot):
        return (pltpu.make_async_copy(srcp_hbm.at[t, w, g], isrc[slot], sem_i[slot]),
                pltpu.make_async_copy(dstp_hbm.at[t, w, g], idst[slot], sem_i[slot]))

    def gather_d(slot, b):
        return pltpu.make_async_copy(y_hbm.at[isrc[slot]], rows[b], sem_g[b])

    def scat_d(slot, b):
        return pltpu.make_async_copy(rows[b], accum.at[idst[slot]], sem_s[b])

    @pl.loop(0, T)
    def _t(t):
        pltpu.async_copy(zeros_hbm, accum.at[pl.ds(s * RPT, RPT)], sem_z)
        for g in range(NBUF):
            for d_ in idx_pair(t, g, g % ISLOT):
                d_.start()
        pltpu.make_async_copy(zeros_hbm, accum.at[pl.ds(s * RPT, RPT)],
                              sem_z).wait()
        plsc.subcore_barrier()

        @pl.loop(0, NCH, step=ISLOT)
        def _g(g0):
            for b in range(ISLOT):
                gg = g0 + b
                rb = b % NBUF

                @pl.when(gg >= NBUF)
                def _w0():
                    scat_d((b - NBUF) % ISLOT, rb).wait()

                for d_ in idx_pair(t, gg, b):
                    d_.wait()
                pltpu.async_copy(y_hbm.at[isrc[b]], rows[rb], sem_g[rb])

                @pl.when(gg + NBUF < NCH)
                def _w1():
                    for d_ in idx_pair(t, gg + NBUF, (b + NBUF) % ISLOT):
                        d_.start()

                @pl.when(gg >= 1)
                def _w2():
                    pb = (b - 1) % NBUF
                    gather_d((b - 1) % ISLOT, pb).wait()
                    pltpu.async_copy(rows[pb], accum.at[idst[(b - 1) % ISLOT]],
                                     sem_s[pb], add=True)

        lb = (NCH - 1) % NBUF
        ls = (NCH - 1) % ISLOT
        gather_d(ls, lb).wait()
        pltpu.async_copy(rows[lb], accum.at[idst[ls]], sem_s[lb], add=True)
        for gg in range(NCH - NBUF, NCH):
            scat_d(gg % ISLOT, gg % NBUF).wait()

        plsc.subcore_barrier()
        pltpu.sync_copy(accum.at[pl.ds(s * RPT, RPT)],
                        out_hbm.at[c, t, pl.ds(s * RPT, RPT)])


def _agg_call(y, srcp, dstp, zrows):
    kern = pl.kernel(
        _agg_body,
        out_type=jax.ShapeDtypeStruct((NC, T, N, D), jnp.bfloat16),
        mesh=_mesh,
        compiler_params=_sc_params,
        scratch_types=[
            pltpu.VMEM_SHARED((N + 16, D), jnp.bfloat16),
            [pltpu.VMEM((K,), jnp.int32) for _ in range(ISLOT)],
            [pltpu.VMEM((K,), jnp.int32) for _ in range(ISLOT)],
            [pltpu.VMEM((K, D), jnp.bfloat16) for _ in range(NBUF)],
            [pltpu.SemaphoreType.DMA for _ in range(ISLOT)],
            [pltpu.SemaphoreType.DMA for _ in range(NBUF)],
            [pltpu.SemaphoreType.DMA for _ in range(NBUF)],
            pltpu.SemaphoreType.DMA,
        ],
    )
    return kern(y, srcp, dstp, zrows)


_RB = 1000
_NRB = T * N // _RB


def _tca_body(x_ref, w_ref, degp_ref, y_ref):
    deg = jnp.sum(degp_ref[...], axis=1) + 1.0
    dinv = jax.lax.rsqrt(deg)
    xw = jnp.dot(x_ref[...], w_ref[...], preferred_element_type=jnp.float32)
    y_ref[...] = (xw * dinv[:, None]).astype(jnp.bfloat16)


@jax.jit
def _tc_a(x2d, W1, degpt):
    return pl.pallas_call(
        _tca_body,
        grid=(_NRB,),
        in_specs=[
            pl.BlockSpec((_RB, D), lambda i: (i, 0)),
            pl.BlockSpec((D, D), lambda i: (0, 0)),
            pl.BlockSpec((_RB, NW), lambda i: (i, 0)),
        ],
        out_specs=pl.BlockSpec((_RB, D), lambda i: (i, 0)),
        out_shape=jax.ShapeDtypeStruct((T * N, D), jnp.bfloat16),
    )(x2d, W1, degpt)


def _tcb_body(aggp_ref, y1_ref, degp_ref, w_ref, b_ref, y2_ref):
    deg = jnp.sum(degp_ref[...], axis=1) + 1.0
    dinv = jax.lax.rsqrt(deg)[:, None]
    a = (aggp_ref[0].astype(jnp.float32) + aggp_ref[1].astype(jnp.float32)
         + y1_ref[...].astype(jnp.float32))
    h1 = jnp.maximum(a * dinv + b_ref[...], 0.0)
    y2_ref[...] = (jnp.dot(h1, w_ref[...], preferred_element_type=jnp.float32)
                   * dinv).astype(jnp.bfloat16)


@jax.jit
def _tc_b(aggp, y1, degpt, W2, b1):
    return pl.pallas_call(
        _tcb_body,
        grid=(_NRB,),
        in_specs=[
            pl.BlockSpec((NC, _RB, D), lambda i: (0, i, 0)),
            pl.BlockSpec((_RB, D), lambda i: (i, 0)),
            pl.BlockSpec((_RB, NW), lambda i: (i, 0)),
            pl.BlockSpec((D, D), lambda i: (0, 0)),
            pl.BlockSpec((1, D), lambda i: (0, 0)),
        ],
        out_specs=pl.BlockSpec((_RB, D), lambda i: (i, 0)),
        out_shape=jax.ShapeDtypeStruct((T * N, D), jnp.bfloat16),
    )(aggp, y1, degpt, W2, b1)


def _tcc_body(aggp_ref, y2_ref, degp_ref, b_ref, emb_ref):
    j = pl.program_id(1)

    @pl.when(j == 0)
    def _():
        emb_ref[...] = jnp.zeros_like(emb_ref)

    deg = jnp.sum(degp_ref[...], axis=1) + 1.0
    dinv = jax.lax.rsqrt(deg)[:, None]
    a = (aggp_ref[0].astype(jnp.float32) + aggp_ref[1].astype(jnp.float32)
         + y2_ref[...].astype(jnp.float32))
    h2 = jnp.maximum(a * dinv + b_ref[...], 0.0)
    emb_ref[0, 0:1, :] += jnp.sum(h2, axis=0, keepdims=True) * (1.0 / N)


@jax.jit
def _tc_c(aggp, y2, degpt, b2):
    npt = N // _RB

    return pl.pallas_call(
        _tcc_body,
        grid=(T, npt),
        in_specs=[
            pl.BlockSpec((NC, _RB, D), lambda t, j: (0, t * npt + j, 0)),
            pl.BlockSpec((_RB, D), lambda t, j: (t * npt + j, 0)),
            pl.BlockSpec((_RB, NW), lambda t, j: (t * npt + j, 0)),
            pl.BlockSpec((1, D), lambda t, j: (0, 0)),
        ],
        out_specs=pl.BlockSpec((1, 8, D), lambda t, j: (t, 0, 0)),
        out_shape=jax.ShapeDtypeStruct((T, 8, D), jnp.float32),
    )(aggp, y2, degpt, b2)


def _gru_body(emb_ref, wih_ref, whh_ref, bih_ref, bhh_ref, wc_ref, bc_ref,
              out_ref):
    h = jnp.zeros((1, GRU_H), jnp.float32)
    for t in range(T):
        xt = emb_ref[pl.ds(t, 1), :]
        gi = jnp.dot(xt, wih_ref[...],
                     preferred_element_type=jnp.float32) + bih_ref[...]
        gh = jnp.dot(h, whh_ref[...],
                     preferred_element_type=jnp.float32) + bhh_ref[...]
        r = jax.nn.sigmoid(gi[:, :GRU_H] + gh[:, :GRU_H])
        z = jax.nn.sigmoid(gi[:, GRU_H:2 * GRU_H] + gh[:, GRU_H:2 * GRU_H])
        n = jnp.tanh(gi[:, 2 * GRU_H:] + r * gh[:, 2 * GRU_H:])
        h = (1.0 - z) * n + z * h
    out_ref[...] = jnp.dot(h, wc_ref[...],
                           preferred_element_type=jnp.float32) + bc_ref[...]


@jax.jit
def _tc_gru(emb, wihT, whhT, bih, bhh, wcT, bc):
    return pl.pallas_call(
        _gru_body,
        out_shape=jax.ShapeDtypeStruct((1, D), jnp.float32),
    )(emb, wihT, whhT, bih, bhh, wcT, bc)


def kernel(x, edge_index, W1, b1, W2, b2, W_ih, W_hh, b_ih, b_hh, Wc, bc):
    ei = edge_index.astype(jnp.int32)
    src = ei[:, 0, :]
    dst = ei[:, 1, :]

    srcg = src + (jnp.arange(T, dtype=jnp.int32) * N)[:, None]
    pad = ((0, 0), (0, 0), (0, EPW_PAD - EPW))
    srcp = jnp.pad(srcg.reshape(T, NW, EPW), pad,
                   constant_values=0).reshape(T, NW, NCH, K)
    dstp = jnp.pad(dst.reshape(T, NW, EPW), pad,
                   constant_values=SINK).reshape(T, NW, NCH, K)
    zrows = jnp.zeros((RPT, D), jnp.bfloat16)

    degp = _sc_deg(dst)
    degpt = degp.reshape(NW, T * N).T

    x2d = x.reshape(T * N, D)
    y1 = _tc_a(x2d, W1, degpt)
    aggp1 = _agg_call(y1, srcp, dstp, zrows)
    y2 = _tc_b(aggp1.reshape(NC, T * N, D), y1, degpt, W2, b1[None, :])
    aggp2 = _agg_call(y2, srcp, dstp, zrows)
    emb = _tc_c(aggp2.reshape(NC, T * N, D), y2, degpt, b2[None, :])[:, 0, :]

    wcT = jnp.zeros((GRU_H, D), jnp.float32).at[:, :10].set(Wc.T)
    bcp = jnp.zeros((1, D), jnp.float32).at[0, :10].set(bc)
    logits = _tc_gru(emb, W_ih.T, W_hh.T, b_ih[None, :], b_hh[None, :],
                     wcT, bcp)
    return logits[:, :10]


# speedup vs baseline: 4.0296x; 1.0173x over previous
import functools

import jax
import jax.numpy as jnp
from jax import lax
from jax.experimental import pallas as pl
from jax.experimental.pallas import tpu as pltpu
from jax.experimental.pallas import tpu_sc as plsc

N = 10000
E = 320000
T = 8
D = 128
GRU_H = 256

NC = 2
NS = 16
NW = NC * NS
EPW = E // NW
K = 512
NCH = 20
EPW_PAD = NCH * K
RPT = N // NS
SINK = N
NBUF = 2
ISLOT = 4

_mesh = plsc.VectorSubcoreMesh(core_axis_name="c", subcore_axis_name="s")
_sc_params = pltpu.CompilerParams(use_tc_tiling_on_sc=False,
                                  needs_layout_passes=False)


def _deg_body(dst_hbm, out_hbm, idx_v, hist_v, sem):
    c = lax.axis_index("c")
    s = lax.axis_index("s")
    w = c * NS + s

    @pl.loop(0, T)
    def _t(t):
        @pl.loop(0, RPT)
        def _z(i):
            hist_v[i, :] = jnp.zeros((16,), jnp.float32)

        pltpu.async_copy(dst_hbm.at[t, pl.ds(w * EPW, EPW)], idx_v, sem).wait()

        ones = jnp.ones((16,), jnp.float32)

        @pl.loop(0, EPW // 16)
        def _e(i):
            idx = idx_v[pl.ds(i * 16, 16)]
            hi = jax.lax.shift_right_logical(idx, 4)
            lo = jnp.bitwise_and(idx, jnp.full((16,), 15, jnp.int32))
            plsc.addupdate_scatter(hist_v, [hi, lo], ones)

        pltpu.async_copy(hist_v, out_hbm.at[w, t], sem).wait()


@jax.jit
def _sc_deg(dst):
    kern = pl.kernel(
        _deg_body,
        out_type=jax.ShapeDtypeStruct((NW, T, RPT, 16), jnp.float32),
        mesh=_mesh,
        compiler_params=_sc_params,
        scratch_types=[
            pltpu.VMEM((EPW,), jnp.int32),
            pltpu.VMEM((RPT, 16), jnp.float32),
            pltpu.SemaphoreType.DMA,
        ],
    )
    return kern(dst)


def _agg_body(y_hbm, srcp_hbm, dstp_hbm, zeros_hbm, out_hbm, accum,
              isrc, idst, rows, sem_i, sem_g, sem_s, sem_z):
    c = lax.axis_index("c")
    s = lax.axis_index("s")
    w = c * NS + s

    def idx_pair(t, g, slot):
        return (pltpu.make_async_copy(srcp_hbm.at[t, w, g], isrc[slot], sem_i[slot]),
                pltpu.make_async_copy(dstp_hbm.at[t, w, g], idst[slot], sem_i[slot]))

    def gather_d(slot, b):
        return pltpu.make_async_copy(y_hbm.at[isrc[slot]], rows[b], sem_g[b])

    def scat_d(slot, b):
        return pltpu.make_async_copy(rows[b], accum.at[idst[slot]], sem_s[b])

    @pl.loop(0, T)
    def _t(t):
        pltpu.async_copy(zeros_hbm, accum.at[pl.ds(s * RPT, RPT)], sem_z)
        for g in range(NBUF):
            for d_ in idx_pair(t, g, g % ISLOT):
                d_.start()
        pltpu.make_async_copy(zeros_hbm, accum.at[pl.ds(s * RPT, RPT)],
                              sem_z).wait()
        plsc.subcore_barrier()

        @pl.loop(0, NCH, step=ISLOT)
        def _g(g0):
            for b in range(ISLOT):
                gg = g0 + b
                rb = b % NBUF

                @pl.when(gg >= NBUF)
                def _w0():
                    scat_d((b - NBUF) % ISLOT, rb).wait()

                for d_ in idx_pair(t, gg, b):
                    d_.wait()
                pltpu.async_copy(y_hbm.at[isrc[b]], rows[rb], sem_g[rb])

                @pl.when(gg + NBUF < NCH)
                def _w1():
                    for d_ in idx_pair(t, gg + NBUF, (b + NBUF) % ISLOT):
                        d_.start()

                @pl.when(gg >= 1)
                def _w2():
                    pb = (b - 1) % NBUF
                    gather_d((b - 1) % ISLOT, pb).wait()
                    pltpu.async_copy(rows[pb], accum.at[idst[(b - 1) % ISLOT]],
                                     sem_s[pb], add=True)

        lb = (NCH - 1) % NBUF
        ls = (NCH - 1) % ISLOT
        gather_d(ls, lb).wait()
        pltpu.async_copy(rows[lb], accum.at[idst[ls]], sem_s[lb], add=True)
        for gg in range(NCH - NBUF, NCH):
            scat_d(gg % ISLOT, gg % NBUF).wait()

        plsc.subcore_barrier()
        pltpu.sync_copy(accum.at[pl.ds(s * RPT, RPT)],
                        out_hbm.at[c, t, pl.ds(s * RPT, RPT)])


def _agg_call(y, srcp, dstp, zrows):
    kern = pl.kernel(
        _agg_body,
        out_type=jax.ShapeDtypeStruct((NC, T, N, D), jnp.bfloat16),
        mesh=_mesh,
        compiler_params=_sc_params,
        scratch_types=[
            pltpu.VMEM_SHARED((N + 16, D), jnp.bfloat16),
            [pltpu.VMEM((K,), jnp.int32) for _ in range(ISLOT)],
            [pltpu.VMEM((K,), jnp.int32) for _ in range(ISLOT)],
            [pltpu.VMEM((K, D), jnp.bfloat16) for _ in range(NBUF)],
            [pltpu.SemaphoreType.DMA for _ in range(ISLOT)],
            [pltpu.SemaphoreType.DMA for _ in range(NBUF)],
            [pltpu.SemaphoreType.DMA for _ in range(NBUF)],
            pltpu.SemaphoreType.DMA,
        ],
    )
    return kern(y, srcp, dstp, zrows)


_RB = 1000
_NRB = T * N // _RB


def _tca_body(x_ref, w_ref, degp_ref, y_ref):
    deg = jnp.sum(degp_ref[...], axis=1) + 1.0
    dinv = jax.lax.rsqrt(deg)
    xw = jnp.dot(x_ref[...], w_ref[...], preferred_element_type=jnp.float32)
    y_ref[...] = (xw * dinv[:, None]).astype(jnp.bfloat16)


@jax.jit
def _tc_a(x2d, W1, degpt):
    return pl.pallas_call(
        _tca_body,
        grid=(_NRB,),
        in_specs=[
            pl.BlockSpec((_RB, D), lambda i: (i, 0)),
            pl.BlockSpec((D, D), lambda i: (0, 0)),
            pl.BlockSpec((_RB, NW), lambda i: (i, 0)),
        ],
        out_specs=pl.BlockSpec((_RB, D), lambda i: (i, 0)),
        out_shape=jax.ShapeDtypeStruct((T * N, D), jnp.bfloat16),
    )(x2d, W1, degpt)


def _tcb_body(aggp_ref, y1_ref, degp_ref, w_ref, b_ref, y2_ref):
    deg = jnp.sum(degp_ref[...], axis=1) + 1.0
    dinv = jax.lax.rsqrt(deg)[:, None]
    a = (aggp_ref[0].astype(jnp.float32) + aggp_ref[1].astype(jnp.float32)
         + y1_ref[...].astype(jnp.float32))
    h1 = jnp.maximum(a * dinv + b_ref[...], 0.0)
    y2_ref[...] = (jnp.dot(h1, w_ref[...], preferred_element_type=jnp.float32)
                   * dinv).astype(jnp.bfloat16)


@jax.jit
def _tc_b(aggp, y1, degpt, W2, b1):
    return pl.pallas_call(
        _tcb_body,
        grid=(_NRB,),
        in_specs=[
            pl.BlockSpec((NC, _RB, D), lambda i: (0, i, 0)),
            pl.BlockSpec((_RB, D), lambda i: (i, 0)),
            pl.BlockSpec((_RB, NW), lambda i: (i, 0)),
            pl.BlockSpec((D, D), lambda i: (0, 0)),
            pl.BlockSpec((1, D), lambda i: (0, 0)),
        ],
        out_specs=pl.BlockSpec((_RB, D), lambda i: (i, 0)),
        out_shape=jax.ShapeDtypeStruct((T * N, D), jnp.bfloat16),
    )(aggp, y1, degpt, W2, b1)


def _tcc_body(aggp_ref, y2_ref, degp_ref, b_ref, emb_ref):
    j = pl.program_id(1)

    @pl.when(j == 0)
    def _():
        emb_ref[...] = jnp.zeros_like(emb_ref)

    deg = jnp.sum(degp_ref[...], axis=1) + 1.0
    dinv = jax.lax.rsqrt(deg)[:, None]
    a = (aggp_ref[0].astype(jnp.float32) + aggp_ref[1].astype(jnp.float32)
         + y2_ref[...].astype(jnp.float32))
    h2 = jnp.maximum(a * dinv + b_ref[...], 0.0)
    emb_ref[0, 0:1, :] += jnp.sum(h2, axis=0, keepdims=True) * (1.0 / N)


@jax.jit
def _tc_c(aggp, y2, degpt, b2):
    npt = N // _RB

    return pl.pallas_call(
        _tcc_body,
        grid=(T, npt),
        in_specs=[
            pl.BlockSpec((NC, _RB, D), lambda t, j: (0, t * npt + j, 0)),
            pl.BlockSpec((_RB, D), lambda t, j: (t * npt + j, 0)),
            pl.BlockSpec((_RB, NW), lambda t, j: (t * npt + j, 0)),
            pl.BlockSpec((1, D), lambda t, j: (0, 0)),
        ],
        out_specs=pl.BlockSpec((1, 8, D), lambda t, j: (t, 0, 0)),
        out_shape=jax.ShapeDtypeStruct((T, 8, D), jnp.float32),
    )(aggp, y2, degpt, b2)


def _gru_body(emb_ref, wih_ref, whh_ref, bih_ref, bhh_ref, wc_ref, bc_ref,
              out_ref):
    h = jnp.zeros((1, GRU_H), jnp.float32)
    for t in range(T):
        xt = emb_ref[pl.ds(t, 1), :]
        gi = jnp.dot(xt, wih_ref[...],
                     preferred_element_type=jnp.float32) + bih_ref[...]
        gh = jnp.dot(h, whh_ref[...],
                     preferred_element_type=jnp.float32) + bhh_ref[...]
        r = jax.nn.sigmoid(gi[:, :GRU_H] + gh[:, :GRU_H])
        z = jax.nn.sigmoid(gi[:, GRU_H:2 * GRU_H] + gh[:, GRU_H:2 * GRU_H])
        n = jnp.tanh(gi[:, 2 * GRU_H:] + r * gh[:, 2 * GRU_H:])
        h = (1.0 - z) * n + z * h
    out_ref[...] = jnp.dot(h, wc_ref[...],
                           preferred_element_type=jnp.float32) + bc_ref[...]


@jax.jit
def _tc_gru(emb, wihT, whhT, bih, bhh, wcT, bc):
    return pl.pallas_call(
        _gru_body,
        out_shape=jax.ShapeDtypeStruct((1, D), jnp.float32),
    )(emb, wihT, whhT, bih, bhh, wcT, bc)


def kernel(x, edge_index, W1, b1, W2, b2, W_ih, W_hh, b_ih, b_hh, Wc, bc):
    ei = edge_index.astype(jnp.int32)
    src = ei[:, 0, :]
    dst = ei[:, 1, :]

    srcg = src + (jnp.arange(T, dtype=jnp.int32) * N)[:, None]
    pad = ((0, 0), (0, 0), (0, EPW_PAD - EPW))
    srcp = jnp.pad(srcg.reshape(T, NW, EPW), pad,
                   constant_values=0).reshape(T, NW, NCH, K)
    dstp = jnp.pad(dst.reshape(T, NW, EPW), pad,
                   constant_values=SINK).reshape(T, NW, NCH, K)
    zrows = jnp.zeros((RPT, D), jnp.bfloat16)

    degp = _sc_deg(dst)
    degpt = degp.reshape(NW, T * N).T

    x2d = x.reshape(T * N, D)
    y1 = _tc_a(x2d, W1, degpt)
    aggp1 = _agg_call(y1, srcp, dstp, zrows)
    y2 = _tc_b(aggp1.reshape(NC, T * N, D), y1, degpt, W2, b1[None, :])
    aggp2 = _agg_call(y2, srcp, dstp, zrows)
    emb = _tc_c(aggp2.reshape(NC, T * N, D), y2, degpt, b2[None, :])[:, 0, :]

    wcT = jnp.zeros((GRU_H, D), jnp.float32).at[:, :10].set(Wc.T)
    bcp = jnp.zeros((1, D), jnp.float32).at[0, :10].set(bc)
    logits = _tc_gru(emb, W_ih.T, W_hh.T, b_ih[None, :], b_hh[None, :],
                     wcT, bcp)
    return logits[:, :10]


# speedup vs baseline: 4.0412x; 1.0029x over previous
import functools

import jax
import jax.numpy as jnp
from jax import lax
from jax.experimental import pallas as pl
from jax.experimental.pallas import tpu as pltpu
from jax.experimental.pallas import tpu_sc as plsc

N = 10000
E = 320000
T = 8
D = 128
GRU_H = 256

NC = 2
NS = 16
NW = NC * NS
EPW = E // NW
K = 640
NCH = 16
EPW_PAD = NCH * K
RPT = N // NS
SINK = N
NBUF = 2
ISLOT = 4

_mesh = plsc.VectorSubcoreMesh(core_axis_name="c", subcore_axis_name="s")
_sc_params = pltpu.CompilerParams(use_tc_tiling_on_sc=False,
                                  needs_layout_passes=False)


def _deg_body(dst_hbm, out_hbm, idx_v, hist_v, sem):
    c = lax.axis_index("c")
    s = lax.axis_index("s")
    w = c * NS + s

    @pl.loop(0, T)
    def _t(t):
        @pl.loop(0, RPT)
        def _z(i):
            hist_v[i, :] = jnp.zeros((16,), jnp.float32)

        pltpu.async_copy(dst_hbm.at[t, pl.ds(w * EPW, EPW)], idx_v, sem).wait()

        ones = jnp.ones((16,), jnp.float32)

        @pl.loop(0, EPW // 16)
        def _e(i):
            idx = idx_v[pl.ds(i * 16, 16)]
            hi = jax.lax.shift_right_logical(idx, 4)
            lo = jnp.bitwise_and(idx, jnp.full((16,), 15, jnp.int32))
            plsc.addupdate_scatter(hist_v, [hi, lo], ones)

        pltpu.async_copy(hist_v, out_hbm.at[w, t], sem).wait()


@jax.jit
def _sc_deg(dst):
    kern = pl.kernel(
        _deg_body,
        out_type=jax.ShapeDtypeStruct((NW, T, RPT, 16), jnp.float32),
        mesh=_mesh,
        compiler_params=_sc_params,
        scratch_types=[
            pltpu.VMEM((EPW,), jnp.int32),
            pltpu.VMEM((RPT, 16), jnp.float32),
            pltpu.SemaphoreType.DMA,
        ],
    )
    return kern(dst)


def _agg_body(y_hbm, srcp_hbm, dstp_hbm, zeros_hbm, out_hbm, accum,
              isrc, idst, rows, sem_i, sem_g, sem_s, sem_z):
    c = lax.axis_index("c")
    s = lax.axis_index("s")
    w = c * NS + s

    def idx_pair(t, g, slot):
        return (pltpu.make_async_copy(srcp_hbm.at[t, w, g], isrc[slot], sem_i[slot]),
                pltpu.make_async_copy(dstp_hbm.at[t, w, g], idst[slot], sem_i[slot]))

    def gather_d(slot, b):
        return pltpu.make_async_copy(y_hbm.at[isrc[slot]], rows[b], sem_g[b])

    def scat_d(slot, b):
        return pltpu.make_async_copy(rows[b], accum.at[idst[slot]], sem_s[b])

    @pl.loop(0, T)
    def _t(t):
        pltpu.async_copy(zeros_hbm, accum.at[pl.ds(s * RPT, RPT)], sem_z)
        for g in range(NBUF):
            for d_ in idx_pair(t, g, g % ISLOT):
                d_.start()
        pltpu.make_async_copy(zeros_hbm, accum.at[pl.ds(s * RPT, RPT)],
                              sem_z).wait()
        plsc.subcore_barrier()

        @pl.loop(0, NCH, step=ISLOT)
        def _g(g0):
            for b in range(ISLOT):
                gg = g0 + b
                rb = b % NBUF

                @pl.when(gg >= NBUF)
                def _w0():
                    scat_d((b - NBUF) % ISLOT, rb).wait()

                for d_ in idx_pair(t, gg, b):
                    d_.wait()
                pltpu.async_copy(y_hbm.at[isrc[b]], rows[rb], sem_g[rb])

                @pl.when(gg + NBUF < NCH)
                def _w1():
                    for d_ in idx_pair(t, gg + NBUF, (b + NBUF) % ISLOT):
                        d_.start()

                @pl.when(gg >= 1)
                def _w2():
                    pb = (b - 1) % NBUF
                    gather_d((b - 1) % ISLOT, pb).wait()
                    pltpu.async_copy(rows[pb], accum.at[idst[(b - 1) % ISLOT]],
                                     sem_s[pb], add=True)

        lb = (NCH - 1) % NBUF
        ls = (NCH - 1) % ISLOT
        gather_d(ls, lb).wait()
        pltpu.async_copy(rows[lb], accum.at[idst[ls]], sem_s[lb], add=True)
        for gg in range(NCH - NBUF, NCH):
            scat_d(gg % ISLOT, gg % NBUF).wait()

        plsc.subcore_barrier()
        pltpu.sync_copy(accum.at[pl.ds(s * RPT, RPT)],
                        out_hbm.at[c, t, pl.ds(s * RPT, RPT)])


def _agg_call(y, srcp, dstp, zrows):
    kern = pl.kernel(
        _agg_body,
        out_type=jax.ShapeDtypeStruct((NC, T, N, D), jnp.bfloat16),
        mesh=_mesh,
        compiler_params=_sc_params,
        scratch_types=[
            pltpu.VMEM_SHARED((N + 16, D), jnp.bfloat16),
            [pltpu.VMEM((K,), jnp.int32) for _ in range(ISLOT)],
            [pltpu.VMEM((K,), jnp.int32) for _ in range(ISLOT)],
            [pltpu.VMEM((K, D), jnp.bfloat16) for _ in range(NBUF)],
            [pltpu.SemaphoreType.DMA for _ in range(ISLOT)],
            [pltpu.SemaphoreType.DMA for _ in range(NBUF)],
            [pltpu.SemaphoreType.DMA for _ in range(NBUF)],
            pltpu.SemaphoreType.DMA,
        ],
    )
    return kern(y, srcp, dstp, zrows)


_RB = 1000
_NRB = T * N // _RB


def _tca_body(x_ref, w_ref, degp_ref, y_ref):
    deg = jnp.sum(degp_ref[...], axis=1) + 1.0
    dinv = jax.lax.rsqrt(deg)
    xw = jnp.dot(x_ref[...], w_ref[...], preferred_element_type=jnp.float32)
    y_ref[...] = (xw * dinv[:, None]).astype(jnp.bfloat16)


@jax.jit
def _tc_a(x2d, W1, degpt):
    return pl.pallas_call(
        _tca_body,
        grid=(_NRB,),
        in_specs=[
            pl.BlockSpec((_RB, D), lambda i: (i, 0)),
            pl.BlockSpec((D, D), lambda i: (0, 0)),
            pl.BlockSpec((_RB, NW), lambda i: (i, 0)),
        ],
        out_specs=pl.BlockSpec((_RB, D), lambda i: (i, 0)),
        out_shape=jax.ShapeDtypeStruct((T * N, D), jnp.bfloat16),
    )(x2d, W1, degpt)


def _tcb_body(aggp_ref, y1_ref, degp_ref, w_ref, b_ref, y2_ref):
    deg = jnp.sum(degp_ref[...], axis=1) + 1.0
    dinv = jax.lax.rsqrt(deg)[:, None]
    a = (aggp_ref[0].astype(jnp.float32) + aggp_ref[1].astype(jnp.float32)
         + y1_ref[...].astype(jnp.float32))
    h1 = jnp.maximum(a * dinv + b_ref[...], 0.0)
    y2_ref[...] = (jnp.dot(h1, w_ref[...], preferred_element_type=jnp.float32)
                   * dinv).astype(jnp.bfloat16)


@jax.jit
def _tc_b(aggp, y1, degpt, W2, b1):
    return pl.pallas_call(
        _tcb_body,
        grid=(_NRB,),
        in_specs=[
            pl.BlockSpec((NC, _RB, D), lambda i: (0, i, 0)),
            pl.BlockSpec((_RB, D), lambda i: (i, 0)),
            pl.BlockSpec((_RB, NW), lambda i: (i, 0)),
            pl.BlockSpec((D, D), lambda i: (0, 0)),
            pl.BlockSpec((1, D), lambda i: (0, 0)),
        ],
        out_specs=pl.BlockSpec((_RB, D), lambda i: (i, 0)),
        out_shape=jax.ShapeDtypeStruct((T * N, D), jnp.bfloat16),
    )(aggp, y1, degpt, W2, b1)


def _tcc_body(aggp_ref, y2_ref, degp_ref, b_ref, emb_ref):
    j = pl.program_id(1)

    @pl.when(j == 0)
    def _():
        emb_ref[...] = jnp.zeros_like(emb_ref)

    deg = jnp.sum(degp_ref[...], axis=1) + 1.0
    dinv = jax.lax.rsqrt(deg)[:, None]
    a = (aggp_ref[0].astype(jnp.float32) + aggp_ref[1].astype(jnp.float32)
         + y2_ref[...].astype(jnp.float32))
    h2 = jnp.maximum(a * dinv + b_ref[...], 0.0)
    emb_ref[0, 0:1, :] += jnp.sum(h2, axis=0, keepdims=True) * (1.0 / N)


@jax.jit
def _tc_c(aggp, y2, degpt, b2):
    npt = N // _RB

    return pl.pallas_call(
        _tcc_body,
        grid=(T, npt),
        in_specs=[
            pl.BlockSpec((NC, _RB, D), lambda t, j: (0, t * npt + j, 0)),
            pl.BlockSpec((_RB, D), lambda t, j: (t * npt + j, 0)),
            pl.BlockSpec((_RB, NW), lambda t, j: (t * npt + j, 0)),
            pl.BlockSpec((1, D), lambda t, j: (0, 0)),
        ],
        out_specs=pl.BlockSpec((1, 8, D), lambda t, j: (t, 0, 0)),
        out_shape=jax.ShapeDtypeStruct((T, 8, D), jnp.float32),
    )(aggp, y2, degpt, b2)


def _gru_body(emb_ref, wih_ref, whh_ref, bih_ref, bhh_ref, wc_ref, bc_ref,
              out_ref):
    h = jnp.zeros((1, GRU_H), jnp.float32)
    for t in range(T):
        xt = emb_ref[pl.ds(t, 1), :]
        gi = jnp.dot(xt, wih_ref[...],
                     preferred_element_type=jnp.float32) + bih_ref[...]
        gh = jnp.dot(h, whh_ref[...],
                     preferred_element_type=jnp.float32) + bhh_ref[...]
        r = jax.nn.sigmoid(gi[:, :GRU_H] + gh[:, :GRU_H])
        z = jax.nn.sigmoid(gi[:, GRU_H:2 * GRU_H] + gh[:, GRU_H:2 * GRU_H])
        n = jnp.tanh(gi[:, 2 * GRU_H:] + r * gh[:, 2 * GRU_H:])
        h = (1.0 - z) * n + z * h
    out_ref[...] = jnp.dot(h, wc_ref[...],
                           preferred_element_type=jnp.float32) + bc_ref[...]


@jax.jit
def _tc_gru(emb, wihT, whhT, bih, bhh, wcT, bc):
    return pl.pallas_call(
        _gru_body,
        out_shape=jax.ShapeDtypeStruct((1, D), jnp.float32),
    )(emb, wihT, whhT, bih, bhh, wcT, bc)


def kernel(x, edge_index, W1, b1, W2, b2, W_ih, W_hh, b_ih, b_hh, Wc, bc):
    ei = edge_index.astype(jnp.int32)
    src = ei[:, 0, :]
    dst = ei[:, 1, :]

    srcg = src + (jnp.arange(T, dtype=jnp.int32) * N)[:, None]
    pad = ((0, 0), (0, 0), (0, EPW_PAD - EPW))
    srcp = jnp.pad(srcg.reshape(T, NW, EPW), pad,
                   constant_values=0).reshape(T, NW, NCH, K)
    dstp = jnp.pad(dst.reshape(T, NW, EPW), pad,
                   constant_values=SINK).reshape(T, NW, NCH, K)
    zrows = jnp.zeros((RPT, D), jnp.bfloat16)

    degp = _sc_deg(dst)
    degpt = degp.reshape(NW, T * N).T

    x2d = x.reshape(T * N, D)
    y1 = _tc_a(x2d, W1, degpt)
    aggp1 = _agg_call(y1, srcp, dstp, zrows)
    y2 = _tc_b(aggp1.reshape(NC, T * N, D), y1, degpt, W2, b1[None, :])
    aggp2 = _agg_call(y2, srcp, dstp, zrows)
    emb = _tc_c(aggp2.reshape(NC, T * N, D), y2, degpt, b2[None, :])[:, 0, :]

    wcT = jnp.zeros((GRU_H, D), jnp.float32).at[:, :10].set(Wc.T)
    bcp = jnp.zeros((1, D), jnp.float32).at[0, :10].set(bc)
    logits = _tc_gru(emb, W_ih.T, W_hh.T, b_ih[None, :], b_hh[None, :],
                     wcT, bcp)
    return logits[:, :10]
